# baseline (speedup 1.0000x reference)
import functools

import jax
import jax.numpy as jnp
from jax import lax
from jax.experimental import pallas as pl
from jax.experimental.pallas import tpu as pltpu
from jax.experimental.pallas import tpu_sc as plsc

N = 10000
E = 320000
H = 128
NC = 2
NS = 16
NW = NC * NS
EPT = E // NW
CH = 80
NCHUNK = EPT // CH
RSTRIDE = 624
RWIN = 640

SCH = 128
NB = 3
NI = 3
SNCH = 81
NIDX = SNCH + NI
EPAD = SNCH * SCH
NP = 10008
SRSTRIDE = 624
SRWIN = 648

_mesh = functools.partial(
    plsc.VectorSubcoreMesh,
    core_axis_name="c", subcore_axis_name="s", num_cores=NC, num_subcores=NS,
)


def _deg_body(dst_hbm, z16_hbm, out_hbm, idx_v, ones_v, acc_sh, sem):
    c = lax.axis_index("c")
    s = lax.axis_index("s")
    r0 = pl.multiple_of(s * RSTRIDE, 8)
    pltpu.sync_copy(z16_hbm.at[pl.ds(r0, RWIN)],
                    acc_sh.at[pl.ds(r0, RWIN)])

    def fill(i, _):
        ones_v[i, :] = jnp.ones((16,), jnp.float32)
        return 0
    lax.fori_loop(0, CH, fill, 0)
    plsc.subcore_barrier()

    base = (c * NS + s) * EPT

    def body(k, _):
        off = pl.multiple_of(base + k * CH, 8)
        pltpu.sync_copy(dst_hbm.at[pl.ds(off, CH)], idx_v)
        pltpu.sync_copy(ones_v, acc_sh.at[idx_v], add=True)
        return 0
    lax.fori_loop(0, NCHUNK, body, 0)
    plsc.subcore_barrier()
    pltpu.sync_copy(acc_sh.at[pl.ds(r0, RWIN)],
                    out_hbm.at[c, pl.ds(r0, RWIN)])


def _deg_partials(dst, z16):
    k = pl.kernel(
        _deg_body,
        out_type=jax.ShapeDtypeStruct((NC, N, 16), jnp.float32),
        mesh=_mesh(),
        scratch_types=[
            pltpu.VMEM((CH,), jnp.int32),
            pltpu.VMEM((CH, 16), jnp.float32),
            pltpu.VMEM_SHARED((N, 16), jnp.float32),
            pltpu.SemaphoreType.DMA,
        ],
    )
    return k(dst, z16)


def _spmm_body(vn_hbm, sd_hbm, z_hbm, out_hbm,
               ib0, ib1, ib2, rows0, rows1, rows2, acc_sh,
               isem0, isem1, isem2, gsem0, gsem1, gsem2):
    c = lax.axis_index("c")
    s = lax.axis_index("s")
    w = c * NS + s
    r0 = pl.multiple_of(s * SRSTRIDE, 8)

    ib = (ib0, ib1, ib2)
    rows = (rows0, rows1, rows2)
    isem = (isem0, isem1, isem2)
    gsem = (gsem0, gsem1, gsem2)

    def fire_i(k, sl):
        pltpu.async_copy(sd_hbm.at[w, k], ib[sl % NI], isem[sl % NI])

    def wait_i(k, sl):
        pltpu.make_async_copy(sd_hbm.at[w, k], ib[sl % NI],
                              isem[sl % NI]).wait()

    def fire_g(sl):
        pltpu.async_copy(vn_hbm.at[ib[sl % NI].at[0]], rows[sl % NB],
                         gsem[sl % NB])

    def wait_g(sl):
        pltpu.make_async_copy(vn_hbm.at[ib[sl % NI].at[0]], rows[sl % NB],
                              gsem[sl % NB]).wait()

    def sync_s(sl):
        pltpu.sync_copy(rows[sl % NB], acc_sh.at[ib[sl % NI].at[1]],
                        add=True)

    for k in range(NI):
        fire_i(k, k)
    pltpu.sync_copy(z_hbm.at[pl.ds(r0, SRWIN)],
                    acc_sh.at[pl.ds(r0, SRWIN)])
    plsc.subcore_barrier()
    wait_i(0, 0)
    fire_g(0)
    wait_i(1, 1)
    fire_g(1)

    def body(j, _):
        for b in range(NB):
            k = j * NB + b
            wait_g(b)
            wait_i(k + 2, b + 2)
            fire_g(b + 2)
            sync_s(b)
            fire_i(k + NI, b)
        return 0
    lax.fori_loop(0, SNCH // NB, body, 0)
    wait_g(SNCH)
    wait_g(SNCH + 1)
    wait_i(SNCH + 2, SNCH + 2)

    plsc.subcore_barrier()
    pltpu.sync_copy(acc_sh.at[pl.ds(r0, SRWIN)],
                    out_hbm.at[c, pl.ds(r0, SRWIN)])


def _spmm_partials(vn, srcdst, z128p):
    k = pl.kernel(
        _spmm_body,
        out_type=jax.ShapeDtypeStruct((NC, NP, H), jnp.float32),
        mesh=_mesh(),
        scratch_types=(
            [pltpu.VMEM((2, SCH), jnp.int32)] * NI
            + [pltpu.VMEM((SCH, H), jnp.float32)] * NB
            + [pltpu.VMEM_SHARED((NP, H), jnp.float32)]
            + [pltpu.SemaphoreType.DMA] * (NI + NB)
        ),
        name="cheb_spmm",
    )
    return k(vn, srcdst, z128p)


_RB = 1000
_GRID = N // _RB


def _embed_body(h_ref, degp_ref, emb_ref, x_ref, vn_ref, dis_ref):
    hcol = h_ref[...]
    cols = lax.broadcasted_iota(jnp.int32, (1, 128), 1)
    onehot = (hcol == cols).astype(jnp.float32)
    x = jnp.dot(onehot, emb_ref[...], preferred_element_type=jnp.float32)
    deg = degp_ref[0, :, 0:1] + degp_ref[1, :, 0:1]
    dis = lax.rsqrt(jnp.maximum(deg, 1.0))
    x_ref[...] = x
    vn_ref[...] = x * dis
    dis_ref[...] = dis


def _embed(h2, degp, emb_pad):
    return pl.pallas_call(
        _embed_body,
        grid=(_GRID,),
        in_specs=[
            pl.BlockSpec((_RB, 1), lambda i: (i, 0)),
            pl.BlockSpec((NC, _RB, 16), lambda i: (0, i, 0)),
            pl.BlockSpec((128, 128), lambda i: (0, 0)),
        ],
        out_specs=[
            pl.BlockSpec((_RB, H), lambda i: (i, 0)),
            pl.BlockSpec((_RB, H), lambda i: (i, 0)),
            pl.BlockSpec((_RB, 1), lambda i: (i, 0)),
        ],
        out_shape=[
            jax.ShapeDtypeStruct((N, H), jnp.float32),
            jax.ShapeDtypeStruct((N, H), jnp.float32),
            jax.ShapeDtypeStruct((N, 1), jnp.float32),
        ],
    )(h2, degp, emb_pad)


def _mid_body(agg_ref, dis_ref, x1_ref, vn1_ref):
    ssum = agg_ref[0] + agg_ref[1]
    dis = dis_ref[...]
    x1 = -(ssum * dis)
    x1_ref[...] = x1
    vn1_ref[...] = x1 * dis


def _mid(agg, dis):
    return pl.pallas_call(
        _mid_body,
        grid=(_GRID,),
        in_specs=[
            pl.BlockSpec((NC, _RB, H), lambda i: (0, i, 0)),
            pl.BlockSpec((_RB, 1), lambda i: (i, 0)),
        ],
        out_specs=[
            pl.BlockSpec((_RB, H), lambda i: (i, 0)),
            pl.BlockSpec((_RB, H), lambda i: (i, 0)),
        ],
        out_shape=[
            jax.ShapeDtypeStruct((N, H), jnp.float32),
            jax.ShapeDtypeStruct((N, H), jnp.float32),
        ],
    )(agg, dis)


def _layer_body(agg_ref, x_ref, x1_ref, dis_ref, w_ref, b_ref,
                xn_ref, vnn_ref):
    dis = dis_ref[...]
    x = x_ref[...]
    ssum = agg_ref[0] + agg_ref[1]
    x2 = -2.0 * (ssum * dis) - x
    out = jnp.dot(x, w_ref[0], preferred_element_type=jnp.float32)
    out = out + jnp.dot(x1_ref[...], w_ref[1],
                        preferred_element_type=jnp.float32)
    out = out + jnp.dot(x2, w_ref[2], preferred_element_type=jnp.float32)
    out = out + b_ref[...]
    out = jnp.maximum(out, 0.0)
    xn = x + out
    xn_ref[...] = xn
    vnn_ref[...] = xn * dis


def _layer(agg, x, x1, dis, wf, bf):
    return pl.pallas_call(
        _layer_body,
        grid=(_GRID,),
        in_specs=[
            pl.BlockSpec((NC, _RB, H), lambda i: (0, i, 0)),
            pl.BlockSpec((_RB, H), lambda i: (i, 0)),
            pl.BlockSpec((_RB, H), lambda i: (i, 0)),
            pl.BlockSpec((_RB, 1), lambda i: (i, 0)),
            pl.BlockSpec((3, H, H), lambda i: (0, 0, 0)),
            pl.BlockSpec((1, H), lambda i: (0, 0)),
        ],
        out_specs=[
            pl.BlockSpec((_RB, H), lambda i: (i, 0)),
            pl.BlockSpec((_RB, H), lambda i: (i, 0)),
        ],
        out_shape=[
            jax.ShapeDtypeStruct((N, H), jnp.float32),
            jax.ShapeDtypeStruct((N, H), jnp.float32),
        ],
    )(agg, x, x1, dis, wf, bf)


def _readout_body(x_ref, w1_ref, b1_ref, w2_ref, b2_ref, w3_ref, b3_ref,
                  o_ref):
    m = jnp.sum(x_ref[...], axis=0, keepdims=True) * (1.0 / N)
    hg = jnp.dot(m, w1_ref[...], preferred_element_type=jnp.float32)
    hg = jnp.maximum(hg + b1_ref[...], 0.0)
    hg = jnp.dot(hg, w2_ref[...], preferred_element_type=jnp.float32)
    hg = jnp.maximum(hg + b2_ref[...], 0.0)
    hg = jnp.dot(hg, w3_ref[...], preferred_element_type=jnp.float32)
    o_ref[...] = hg + b3_ref[...]


def _readout(x, w1, b1, w2, b2, w3, b3):
    full = lambda shp: pl.BlockSpec(shp, lambda: tuple(0 for _ in shp))
    return pl.pallas_call(
        _readout_body,
        in_specs=[
            full((N, H)),
            full((H, H)), full((1, H)),
            full((H, H)), full((1, H)),
            full((H, H)), full((1, H)),
        ],
        out_specs=full((1, H)),
        out_shape=jax.ShapeDtypeStruct((1, H), jnp.float32),
    )(x, w1, b1, w2, b2, w3, b3)


def kernel(params, h, edge_index, e):
    src = edge_index[0]
    dst = edge_index[1]
    srcp = jnp.pad(src.reshape(NW, EPT), ((0, 0), (0, EPAD - EPT)),
                   constant_values=0).reshape(NW, SNCH, SCH)
    dstp = jnp.pad(dst.reshape(NW, EPT), ((0, 0), (0, EPAD - EPT)),
                   constant_values=N).reshape(NW, SNCH, SCH)
    srcdst = jnp.pad(jnp.stack([srcp, dstp], axis=2),
                     ((0, 0), (0, NIDX - SNCH), (0, 0), (0, 0)))
    h2 = h.reshape(N, 1)
    emb_pad = jnp.pad(params["emb"], ((0, 128 - params["emb"].shape[0]),
                                      (0, 0)))
    z16 = jnp.zeros((N, 16), jnp.float32)
    z128p = jnp.zeros((NP, H), jnp.float32)

    degp = _deg_partials(dst, z16)
    x, vn, dis = _embed(h2, degp, emb_pad)

    for lp in params["layers"]:
        wf = lp["W"] * lp["gamma"][None, None, :]
        bf = (lp["b"] * lp["gamma"] + lp["beta"]).reshape(1, H)
        agg0 = _spmm_partials(vn, srcdst, z128p)
        x1, vn1 = _mid(agg0, dis)
        agg1 = _spmm_partials(vn1, srcdst, z128p)
        x, vn = _layer(agg1, x, x1, dis, wf, bf)

    mlp = params["mlp"]
    w1 = jnp.pad(mlp[0]["W"], ((0, 0), (0, H - 64)))
    b1 = jnp.pad(mlp[0]["b"], (0, H - 64)).reshape(1, H)
    w2 = jnp.pad(mlp[1]["W"], ((0, H - 64), (0, H - 32)))
    b2 = jnp.pad(mlp[1]["b"], (0, H - 32)).reshape(1, H)
    w3 = jnp.pad(mlp[2]["W"], ((0, H - 32), (0, H - 1)))
    b3 = jnp.pad(mlp[2]["b"], (0, H - 1)).reshape(1, H)
    out = _readout(x, w1, b1, w2, b2, w3, b3)
    return out[:, :1]


# speedup vs baseline: 3.1513x; 3.1513x over previous
import functools

import jax
import jax.numpy as jnp
from jax import lax
from jax.experimental import pallas as pl
from jax.experimental.pallas import tpu as pltpu
from jax.experimental.pallas import tpu_sc as plsc

N = 10000
E = 320000
H = 128
NC = 2
NS = 16
NW = NC * NS
EPT = E // NW

SCH = 128
SNCH = -(-EPT // SCH)
NIDX = SNCH + 1
EPAD = SNCH * SCH
NP = 10008
SRSTRIDE = 624
SRWIN = 648

_mesh = functools.partial(
    plsc.VectorSubcoreMesh,
    core_axis_name="c", subcore_axis_name="s", num_cores=NC, num_subcores=NS,
)


def _deg_body(sd_hbm, z16_hbm, out_hbm, ib0, ib1, ones_v, acc_sh,
              isem0, isem1):
    c = lax.axis_index("c")
    s = lax.axis_index("s")
    w = c * NS + s
    r0 = pl.multiple_of(s * SRSTRIDE, 8)
    pltpu.async_copy(sd_hbm.at[w, 0], ib0, isem0)
    pltpu.async_copy(sd_hbm.at[w, 1], ib1, isem1)
    pltpu.sync_copy(z16_hbm.at[pl.ds(r0, SRWIN)],
                    acc_sh.at[pl.ds(r0, SRWIN)])

    def fill(i, _):
        ones_v[i, :] = jnp.ones((16,), jnp.float32)
        return 0
    lax.fori_loop(0, SCH, fill, 0)
    plsc.subcore_barrier()

    ib = (ib0, ib1)
    isem = (isem0, isem1)

    def body(j, _):
        for b in range(2):
            k = j * 2 + b
            p = b
            pltpu.make_async_copy(sd_hbm.at[w, k], ib[p], isem[p]).wait()
            pltpu.sync_copy(ones_v, acc_sh.at[ib[p].at[1]], add=True)
            pltpu.async_copy(sd_hbm.at[w, k + 2], ib[p], isem[p])
        return 0
    lax.fori_loop(0, (SNCH - 1) // 2, body, 0)
    kl = SNCH - 1
    pltpu.make_async_copy(sd_hbm.at[w, kl], ib0, isem0).wait()
    pltpu.sync_copy(ones_v, acc_sh.at[ib0.at[1]], add=True)
    pltpu.make_async_copy(sd_hbm.at[w, kl + 1], ib1, isem1).wait()

    plsc.subcore_barrier()
    pltpu.sync_copy(acc_sh.at[pl.ds(r0, SRWIN)],
                    out_hbm.at[c, pl.ds(r0, SRWIN)])


def _deg_partials(srcdst, z16):
    k = pl.kernel(
        _deg_body,
        out_type=jax.ShapeDtypeStruct((NC, NP, 16), jnp.float32),
        mesh=_mesh(),
        scratch_types=[
            pltpu.VMEM((2, SCH), jnp.int32),
            pltpu.VMEM((2, SCH), jnp.int32),
            pltpu.VMEM((SCH, 16), jnp.float32),
            pltpu.VMEM_SHARED((NP, 16), jnp.float32),
            pltpu.SemaphoreType.DMA,
            pltpu.SemaphoreType.DMA,
        ],
    )
    return k(srcdst, z16)


def _spmm_body(vn_hbm, sd_hbm, z_hbm, out_hbm,
               ib0, ib1, rows0, rows1, acc_sh, isem0, isem1, gsem0, gsem1):
    c = lax.axis_index("c")
    s = lax.axis_index("s")
    w = c * NS + s
    r0 = pl.multiple_of(s * SRSTRIDE, 8)
    pltpu.async_copy(sd_hbm.at[w, 0], ib0, isem0)
    pltpu.async_copy(sd_hbm.at[w, 1], ib1, isem1)
    pltpu.sync_copy(z_hbm.at[pl.ds(r0, SRWIN)],
                    acc_sh.at[pl.ds(r0, SRWIN)])
    plsc.subcore_barrier()

    ib = (ib0, ib1)
    rows = (rows0, rows1)
    isem = (isem0, isem1)
    gsem = (gsem0, gsem1)

    def wait_i(k, p):
        pltpu.make_async_copy(sd_hbm.at[w, k], ib[p], isem[p]).wait()

    def wait_g(p):
        pltpu.make_async_copy(vn_hbm.at[ib[p].at[0]], rows[p],
                              gsem[p]).wait()

    wait_i(0, 0)
    pltpu.async_copy(vn_hbm.at[ib0.at[0]], rows0, gsem0)

    def body(j, _):
        for b in range(2):
            k = j * 2 + b
            p, q = b, 1 - b
            wait_g(p)
            wait_i(k + 1, q)
            pltpu.async_copy(vn_hbm.at[ib[q].at[0]], rows[q], gsem[q])
            pltpu.sync_copy(rows[p], acc_sh.at[ib[p].at[1]], add=True)
            pltpu.async_copy(sd_hbm.at[w, k + 2], ib[p], isem[p])
        return 0
    lax.fori_loop(0, (SNCH - 1) // 2, body, 0)
    kl = SNCH - 1
    wait_g(0)
    pltpu.sync_copy(rows0, acc_sh.at[ib0.at[1]], add=True)
    wait_i(kl + 1, 1)

    plsc.subcore_barrier()
    pltpu.sync_copy(acc_sh.at[pl.ds(r0, SRWIN)],
                    out_hbm.at[c, pl.ds(r0, SRWIN)])


def _spmm_partials(vn, srcdst, z128p):
    k = pl.kernel(
        _spmm_body,
        out_type=jax.ShapeDtypeStruct((NC, NP, H), jnp.float32),
        mesh=_mesh(),
        scratch_types=[
            pltpu.VMEM((2, SCH), jnp.int32),
            pltpu.VMEM((2, SCH), jnp.int32),
            pltpu.VMEM((SCH, H), jnp.float32),
            pltpu.VMEM((SCH, H), jnp.float32),
            pltpu.VMEM_SHARED((NP, H), jnp.float32),
            pltpu.SemaphoreType.DMA,
            pltpu.SemaphoreType.DMA,
            pltpu.SemaphoreType.DMA,
            pltpu.SemaphoreType.DMA,
        ],
    )
    return k(vn, srcdst, z128p)


_RB = 1000
_GRID = N // _RB


def _embed_body(h_ref, degp_ref, emb_ref, x_ref, vn_ref, dis_ref):
    hcol = h_ref[...]
    cols = lax.broadcasted_iota(jnp.int32, (1, 128), 1)
    onehot = (hcol == cols).astype(jnp.float32)
    x = jnp.dot(onehot, emb_ref[...], preferred_element_type=jnp.float32)
    deg = degp_ref[0, :, 0:1] + degp_ref[1, :, 0:1]
    dis = lax.rsqrt(jnp.maximum(deg, 1.0))
    x_ref[...] = x
    vn_ref[...] = x * dis
    dis_ref[...] = dis


def _embed(h2, degp, emb_pad):
    return pl.pallas_call(
        _embed_body,
        grid=(_GRID,),
        in_specs=[
            pl.BlockSpec((_RB, 1), lambda i: (i, 0)),
            pl.BlockSpec((NC, _RB, 16), lambda i: (0, i, 0)),
            pl.BlockSpec((128, 128), lambda i: (0, 0)),
        ],
        out_specs=[
            pl.BlockSpec((_RB, H), lambda i: (i, 0)),
            pl.BlockSpec((_RB, H), lambda i: (i, 0)),
            pl.BlockSpec((_RB, 1), lambda i: (i, 0)),
        ],
        out_shape=[
            jax.ShapeDtypeStruct((N, H), jnp.float32),
            jax.ShapeDtypeStruct((N, H), jnp.float32),
            jax.ShapeDtypeStruct((N, 1), jnp.float32),
        ],
    )(h2, degp, emb_pad)


def _mid_body(agg_ref, dis_ref, x1_ref, vn1_ref):
    ssum = agg_ref[0] + agg_ref[1]
    dis = dis_ref[...]
    x1 = -(ssum * dis)
    x1_ref[...] = x1
    vn1_ref[...] = x1 * dis


def _mid(agg, dis):
    return pl.pallas_call(
        _mid_body,
        grid=(_GRID,),
        in_specs=[
            pl.BlockSpec((NC, _RB, H), lambda i: (0, i, 0)),
            pl.BlockSpec((_RB, 1), lambda i: (i, 0)),
        ],
        out_specs=[
            pl.BlockSpec((_RB, H), lambda i: (i, 0)),
            pl.BlockSpec((_RB, H), lambda i: (i, 0)),
        ],
        out_shape=[
            jax.ShapeDtypeStruct((N, H), jnp.float32),
            jax.ShapeDtypeStruct((N, H), jnp.float32),
        ],
    )(agg, dis)


def _layer_body(agg_ref, x_ref, x1_ref, dis_ref, w_ref, b_ref,
                xn_ref, vnn_ref):
    dis = dis_ref[...]
    x = x_ref[...]
    ssum = agg_ref[0] + agg_ref[1]
    x2 = -2.0 * (ssum * dis) - x
    out = jnp.dot(x, w_ref[0], preferred_element_type=jnp.float32)
    out = out + jnp.dot(x1_ref[...], w_ref[1],
                        preferred_element_type=jnp.float32)
    out = out + jnp.dot(x2, w_ref[2], preferred_element_type=jnp.float32)
    out = out + b_ref[...]
    out = jnp.maximum(out, 0.0)
    xn = x + out
    xn_ref[...] = xn
    vnn_ref[...] = xn * dis


def _layer(agg, x, x1, dis, wf, bf):
    return pl.pallas_call(
        _layer_body,
        grid=(_GRID,),
        in_specs=[
            pl.BlockSpec((NC, _RB, H), lambda i: (0, i, 0)),
            pl.BlockSpec((_RB, H), lambda i: (i, 0)),
            pl.BlockSpec((_RB, H), lambda i: (i, 0)),
            pl.BlockSpec((_RB, 1), lambda i: (i, 0)),
            pl.BlockSpec((3, H, H), lambda i: (0, 0, 0)),
            pl.BlockSpec((1, H), lambda i: (0, 0)),
        ],
        out_specs=[
            pl.BlockSpec((_RB, H), lambda i: (i, 0)),
            pl.BlockSpec((_RB, H), lambda i: (i, 0)),
        ],
        out_shape=[
            jax.ShapeDtypeStruct((N, H), jnp.float32),
            jax.ShapeDtypeStruct((N, H), jnp.float32),
        ],
    )(agg, x, x1, dis, wf, bf)


def _readout_body(x_ref, w1_ref, b1_ref, w2_ref, b2_ref, w3_ref, b3_ref,
                  o_ref):
    m = jnp.sum(x_ref[...], axis=0, keepdims=True) * (1.0 / N)
    hg = jnp.dot(m, w1_ref[...], preferred_element_type=jnp.float32)
    hg = jnp.maximum(hg + b1_ref[...], 0.0)
    hg = jnp.dot(hg, w2_ref[...], preferred_element_type=jnp.float32)
    hg = jnp.maximum(hg + b2_ref[...], 0.0)
    hg = jnp.dot(hg, w3_ref[...], preferred_element_type=jnp.float32)
    o_ref[...] = hg + b3_ref[...]


def _readout(x, w1, b1, w2, b2, w3, b3):
    full = lambda shp: pl.BlockSpec(shp, lambda: tuple(0 for _ in shp))
    return pl.pallas_call(
        _readout_body,
        in_specs=[
            full((N, H)),
            full((H, H)), full((1, H)),
            full((H, H)), full((1, H)),
            full((H, H)), full((1, H)),
        ],
        out_specs=full((1, H)),
        out_shape=jax.ShapeDtypeStruct((1, H), jnp.float32),
    )(x, w1, b1, w2, b2, w3, b3)


def kernel(params, h, edge_index, e):
    src = edge_index[0]
    dst = edge_index[1]
    srcp = jnp.pad(src.reshape(NW, EPT), ((0, 0), (0, EPAD - EPT)),
                   constant_values=0).reshape(NW, SNCH, SCH)
    dstp = jnp.pad(dst.reshape(NW, EPT), ((0, 0), (0, EPAD - EPT)),
                   constant_values=N).reshape(NW, SNCH, SCH)
    srcdst = jnp.pad(jnp.stack([srcp, dstp], axis=2),
                     ((0, 0), (0, NIDX - SNCH), (0, 0), (0, 0)))
    h2 = h.reshape(N, 1)
    emb_pad = jnp.pad(params["emb"], ((0, 128 - params["emb"].shape[0]),
                                      (0, 0)))
    z16 = jnp.zeros((NP, 16), jnp.float32)
    z128p = jnp.zeros((NP, H), jnp.float32)

    degp = _deg_partials(srcdst, z16)
    x, vn, dis = _embed(h2, degp, emb_pad)

    for lp in params["layers"]:
        wf = lp["W"] * lp["gamma"][None, None, :]
        bf = (lp["b"] * lp["gamma"] + lp["beta"]).reshape(1, H)
        agg0 = _spmm_partials(vn, srcdst, z128p)
        x1, vn1 = _mid(agg0, dis)
        agg1 = _spmm_partials(vn1, srcdst, z128p)
        x, vn = _layer(agg1, x, x1, dis, wf, bf)

    mlp = params["mlp"]
    w1 = jnp.pad(mlp[0]["W"], ((0, 0), (0, H - 64)))
    b1 = jnp.pad(mlp[0]["b"], (0, H - 64)).reshape(1, H)
    w2 = jnp.pad(mlp[1]["W"], ((0, H - 64), (0, H - 32)))
    b2 = jnp.pad(mlp[1]["b"], (0, H - 32)).reshape(1, H)
    w3 = jnp.pad(mlp[2]["W"], ((0, H - 32), (0, H - 1)))
    b3 = jnp.pad(mlp[2]["b"], (0, H - 1)).reshape(1, H)
    out = _readout(x, w1, b1, w2, b2, w3, b3)
    return out[:, :1]
